# baseline (speedup 1.0000x reference)
import jax
import jax.numpy as jnp
from jax.experimental import pallas as pl
from jax.experimental.pallas import tpu as pltpu


def _copy_kernel(x_ref, o_ref, sem):
    copy = pltpu.make_async_copy(x_ref, o_ref, sem)
    copy.start()
    copy.wait()


def kernel(input):
    return pl.pallas_call(
        _copy_kernel,
        out_shape=jax.ShapeDtypeStruct(input.shape, input.dtype),
        in_specs=[pl.BlockSpec(memory_space=pl.ANY)],
        out_specs=pl.BlockSpec(memory_space=pl.ANY),
        scratch_shapes=[pltpu.SemaphoreType.DMA],
    )(input)


# speedup vs baseline: 9.4312x; 9.4312x over previous
import jax
import jax.numpy as jnp
from jax.experimental import pallas as pl
from jax.experimental.pallas import tpu as pltpu


def _copy_kernel(x_ref, o_ref):
    o_ref[...] = x_ref[...]


def kernel(input):
    t, b, s = input.shape
    grid = 4
    return pl.pallas_call(
        _copy_kernel,
        grid=(grid,),
        in_specs=[pl.BlockSpec((t // grid, b, s), lambda i: (i, 0, 0))],
        out_specs=pl.BlockSpec((t // grid, b, s), lambda i: (i, 0, 0)),
        out_shape=jax.ShapeDtypeStruct(input.shape, input.dtype),
        compiler_params=pltpu.CompilerParams(
            dimension_semantics=("parallel",),
        ),
    )(input)


# speedup vs baseline: 15.4059x; 1.6335x over previous
import jax
import jax.numpy as jnp
from jax.experimental import pallas as pl
from jax.experimental.pallas import tpu as pltpu


def _copy_kernel(x_ref, o_ref, buf0, buf1, si0, si1, so0, so1):
    h = x_ref.shape[0] // 2
    in0 = pltpu.make_async_copy(x_ref.at[pl.ds(0, h)], buf0, si0)
    in1 = pltpu.make_async_copy(x_ref.at[pl.ds(h, h)], buf1, si1)
    in0.start()
    in1.start()
    in0.wait()
    out0 = pltpu.make_async_copy(buf0, o_ref.at[pl.ds(0, h)], so0)
    out0.start()
    in1.wait()
    out1 = pltpu.make_async_copy(buf1, o_ref.at[pl.ds(h, h)], so1)
    out1.start()
    out0.wait()
    out1.wait()


def kernel(input):
    t, b, s = input.shape
    return pl.pallas_call(
        _copy_kernel,
        out_shape=jax.ShapeDtypeStruct(input.shape, input.dtype),
        in_specs=[pl.BlockSpec(memory_space=pl.ANY)],
        out_specs=pl.BlockSpec(memory_space=pl.ANY),
        scratch_shapes=[
            pltpu.VMEM((t // 2, b, s), input.dtype),
            pltpu.VMEM((t // 2, b, s), input.dtype),
            pltpu.SemaphoreType.DMA,
            pltpu.SemaphoreType.DMA,
            pltpu.SemaphoreType.DMA,
            pltpu.SemaphoreType.DMA,
        ],
    )(input)
